# baseline (speedup 1.0000x reference)
import jax
import jax.numpy as jnp
from jax import lax
from jax.experimental import pallas as pl
from jax.experimental.pallas import tpu as pltpu
from jax.experimental.pallas import tpu_sc as plsc

B = 16384
D = 128
SD = 32
NC = 2
NS = 16
NW = NC * NS
BPW = B // NW
GCH = 128
NG = BPW // GCH
L = 16
SP = L + 1


def _body(dow_h, time_h, month_h, day_h, dest_h,
          wdow_h, wtime_h, wmonth_h, wday_h, witem_h, wp_h, b_h,
          out_h,
          dest_v, dow_v, time_v, month_v, day_v,
          wdow_v, wtime_v, wmonth_v, wday_v, wp_v, b_v,
          item_v, stage_v, out_v,
          sem_s, sem_d, sem_o, sem_g0, sem_g1, sem_g2, sem_g3):
    wid = lax.axis_index("s") * NC + lax.axis_index("c")
    base = wid * BPW

    gsems = (sem_g0, sem_g1, sem_g2, sem_g3)
    pltpu.sync_copy(dest_h.at[pl.ds(base, GCH)], dest_v.at[0])
    gathers = [pltpu.async_copy(witem_h.at[dest_v.at[0]],
                                item_v.at[pl.ds(0, GCH)], gsems[0])]

    small = [
        pltpu.async_copy(dow_h.at[pl.ds(base, BPW)], dow_v, sem_s),
        pltpu.async_copy(time_h.at[pl.ds(base, BPW)], time_v, sem_s),
        pltpu.async_copy(month_h.at[pl.ds(base, BPW)], month_v, sem_s),
        pltpu.async_copy(day_h.at[pl.ds(base, BPW)], day_v, sem_s),
        pltpu.async_copy(wdow_h, wdow_v, sem_s),
        pltpu.async_copy(wtime_h, wtime_v, sem_s),
        pltpu.async_copy(wmonth_h, wmonth_v, sem_s),
        pltpu.async_copy(wday_h, wday_v, sem_s),
        pltpu.async_copy(wp_h, wp_v, sem_s),
        pltpu.async_copy(b_h, b_v, sem_s),
    ]

    for j in range(1, NG):
        pltpu.sync_copy(dest_h.at[pl.ds(base + j * GCH, GCH)], dest_v.at[j])
        gathers.append(
            pltpu.async_copy(witem_h.at[dest_v.at[j]],
                             item_v.at[pl.ds(j * GCH, GCH)], gsems[j]))

    for c in small:
        c.wait()

    lanes = lax.iota(jnp.int32, L)
    zeros = jnp.zeros((L,), jnp.int32)
    wp = [plsc.load_gather(wp_v, [lanes + L * j, zeros]) for j in range(D // L)]
    bvec = plsc.load_gather(b_v, [zeros])
    tables = (wdow_v, wtime_v, wmonth_v, wday_v)
    idx_refs = (dow_v, time_v, month_v, day_v)
    lanes_sp = lanes * SP

    for seg, (tab, nrows) in enumerate(zip(tables, (7, 24, 12, 31))):
        wpa, wpb = wp[2 * seg], wp[2 * seg + 1]

        def scale_body(r, carry, tab=tab, wpa=wpa, wpb=wpb):
            tab[r, pl.ds(0, L)] = tab[r, pl.ds(0, L)] * wpa
            tab[r, pl.ds(L, L)] = tab[r, pl.ds(L, L)] * wpb
            return carry

        lax.fori_loop(0, nrows, scale_body, 0)

    GPC = GCH // L

    def group_body(g, carry):
        @pl.when(g % GPC == 0)
        def _():
            for j in range(NG):
                @pl.when(g == j * GPC)
                def _(j=j):
                    gathers[j].wait()
        gbase = g * L
        ivecs = [r[pl.ds(gbase, L)] for r in idx_refs]
        for r in range(L):
            row = gbase + r
            acc = None
            for j in range(D // L):
                seg, half = j // 2, (j % 2) * L
                it = item_v[row, pl.ds(L * j, L)]
                us = tables[seg][ivecs[seg][r], pl.ds(half, L)]
                term = it * us
                acc = term if acc is None else acc + term
            stage_v[pl.ds(r * SP, L)] = acc
        cols = [plsc.load_gather(stage_v, [lanes_sp + c]) for c in range(L)]
        while len(cols) > 1:
            cols = [a + b for a, b in zip(cols[::2], cols[1::2])]
        x = cols[0] + bvec
        out_v[pl.ds(gbase, L)] = jnp.where(x >= 0, x, x * 0.01)
        return carry

    lax.fori_loop(0, BPW // L, group_body, 0)

    pltpu.sync_copy(out_v, out_h.at[pl.ds(base, BPW)])


@jax.jit
def _run(dow, time, month, day, dest, W_dow, W_time, W_month, W_day,
         W_item, W_pred, b_pred):
    mesh = plsc.VectorSubcoreMesh(core_axis_name="c", subcore_axis_name="s")
    f = pl.kernel(
        _body,
        out_type=jax.ShapeDtypeStruct((B,), jnp.float32),
        mesh=mesh,
        scratch_types=[
            pltpu.VMEM((NG, GCH), jnp.int32),
            pltpu.VMEM((BPW,), jnp.int32),
            pltpu.VMEM((BPW,), jnp.int32),
            pltpu.VMEM((BPW,), jnp.int32),
            pltpu.VMEM((BPW,), jnp.int32),
            pltpu.VMEM((7, SD), jnp.float32),
            pltpu.VMEM((24, SD), jnp.float32),
            pltpu.VMEM((12, SD), jnp.float32),
            pltpu.VMEM((31, SD), jnp.float32),
            pltpu.VMEM((D, 1), jnp.float32),
            pltpu.VMEM((1,), jnp.float32),
            pltpu.VMEM((BPW, D), jnp.float32),
            pltpu.VMEM((SP * L,), jnp.float32),
            pltpu.VMEM((BPW,), jnp.float32),
            pltpu.SemaphoreType.DMA,
            pltpu.SemaphoreType.DMA,
            pltpu.SemaphoreType.DMA,
            pltpu.SemaphoreType.DMA,
            pltpu.SemaphoreType.DMA,
            pltpu.SemaphoreType.DMA,
            pltpu.SemaphoreType.DMA,
        ],
        compiler_params=pltpu.CompilerParams(needs_layout_passes=False),
    )
    return f(dow, time, month, day, dest, W_dow, W_time, W_month, W_day,
             W_item, W_pred, b_pred)


def kernel(dayofweek, time, month, day, destination, W_dow, W_time,
           W_month, W_day, W_item, W_pred, b_pred):
    return _run(
        dayofweek.astype(jnp.int32), time.astype(jnp.int32),
        month.astype(jnp.int32), day.astype(jnp.int32),
        destination.astype(jnp.int32),
        W_dow, W_time, W_month, W_day, W_item, W_pred, b_pred)


# speedup vs baseline: 1.0211x; 1.0211x over previous
import jax
import jax.numpy as jnp
from jax import lax
from jax.experimental import pallas as pl
from jax.experimental.pallas import tpu as pltpu
from jax.experimental.pallas import tpu_sc as plsc

B = 16384
D = 128
SD = 32
NC = 2
NS = 16
NW = NC * NS
BPW = B // NW
GCH = 128
NG = BPW // GCH
L = 16
SP = L + 1


def _body(dow_h, time_h, month_h, day_h, dest_h,
          wdow_h, wtime_h, wmonth_h, wday_h, witem_h, wp_h, b_h,
          out_h,
          dest_v, dow_v, time_v, month_v, day_v,
          wdow_v, wtime_v, wmonth_v, wday_v, wp_v, b_v,
          item_v, stage_v, out_v,
          sem_s, sem_d, sem_o, sem_g0, sem_g1, sem_g2, sem_g3):
    wid = lax.axis_index("s") * NC + lax.axis_index("c")
    base = wid * BPW

    gsems = (sem_g0, sem_g1, sem_g2, sem_g3)
    pltpu.sync_copy(dest_h.at[pl.ds(base, GCH)], dest_v.at[0])
    gathers = [pltpu.async_copy(witem_h.at[dest_v.at[0]],
                                item_v.at[pl.ds(0, GCH)], gsems[0])]

    small = [
        pltpu.async_copy(dow_h.at[pl.ds(base, BPW)], dow_v, sem_s),
        pltpu.async_copy(time_h.at[pl.ds(base, BPW)], time_v, sem_s),
        pltpu.async_copy(month_h.at[pl.ds(base, BPW)], month_v, sem_s),
        pltpu.async_copy(day_h.at[pl.ds(base, BPW)], day_v, sem_s),
        pltpu.async_copy(wdow_h, wdow_v, sem_s),
        pltpu.async_copy(wtime_h, wtime_v, sem_s),
        pltpu.async_copy(wmonth_h, wmonth_v, sem_s),
        pltpu.async_copy(wday_h, wday_v, sem_s),
        pltpu.async_copy(wp_h, wp_v, sem_s),
        pltpu.async_copy(b_h, b_v, sem_s),
    ]

    for j in range(1, NG):
        pltpu.sync_copy(dest_h.at[pl.ds(base + j * GCH, GCH)], dest_v.at[j])
        gathers.append(
            pltpu.async_copy(witem_h.at[dest_v.at[j]],
                             item_v.at[pl.ds(j * GCH, GCH)], gsems[j]))

    for c in small:
        c.wait()

    lanes = lax.iota(jnp.int32, L)
    zeros = jnp.zeros((L,), jnp.int32)
    wp = [plsc.load_gather(wp_v, [lanes + L * j, zeros]) for j in range(D // L)]
    bvec = plsc.load_gather(b_v, [zeros])
    tables = (wdow_v, wtime_v, wmonth_v, wday_v)
    idx_refs = (dow_v, time_v, month_v, day_v)
    lanes_sp = lanes * SP

    for seg, (tab, nrows) in enumerate(zip(tables, (7, 24, 12, 31))):
        wpa, wpb = wp[2 * seg], wp[2 * seg + 1]

        def scale_body(r, carry, tab=tab, wpa=wpa, wpb=wpb):
            tab[r, pl.ds(0, L)] = tab[r, pl.ds(0, L)] * wpa
            tab[r, pl.ds(L, L)] = tab[r, pl.ds(L, L)] * wpb
            return carry

        lax.fori_loop(0, nrows, scale_body, 0)

    GPC = GCH // L

    def group_body(g, carry):
        @pl.when(g % GPC == 0)
        def _():
            for j in range(NG):
                @pl.when(g == j * GPC)
                def _(j=j):
                    gathers[j].wait()
        gbase = g * L
        ivecs = [r[pl.ds(gbase, L)] for r in idx_refs]
        for rp in range(0, L, 2):
            accs = [None, None]
            for j in range(D // L):
                seg, half = j // 2, (j % 2) * L
                for t in (0, 1):
                    r = rp + t
                    it = item_v[gbase + r, pl.ds(L * j, L)]
                    us = tables[seg][ivecs[seg][r], pl.ds(half, L)]
                    term = it * us
                    accs[t] = term if accs[t] is None else accs[t] + term
            stage_v[pl.ds(rp * SP, L)] = accs[0]
            stage_v[pl.ds((rp + 1) * SP, L)] = accs[1]
        cols = [plsc.load_gather(stage_v, [lanes_sp + c]) for c in range(L)]
        while len(cols) > 1:
            cols = [a + b for a, b in zip(cols[::2], cols[1::2])]
        x = cols[0] + bvec
        out_v[pl.ds(gbase, L)] = jnp.where(x >= 0, x, x * 0.01)
        return carry

    lax.fori_loop(0, BPW // L, group_body, 0)

    pltpu.sync_copy(out_v, out_h.at[pl.ds(base, BPW)])


@jax.jit
def _run(dow, time, month, day, dest, W_dow, W_time, W_month, W_day,
         W_item, W_pred, b_pred):
    mesh = plsc.VectorSubcoreMesh(core_axis_name="c", subcore_axis_name="s")
    f = pl.kernel(
        _body,
        out_type=jax.ShapeDtypeStruct((B,), jnp.float32),
        mesh=mesh,
        scratch_types=[
            pltpu.VMEM((NG, GCH), jnp.int32),
            pltpu.VMEM((BPW,), jnp.int32),
            pltpu.VMEM((BPW,), jnp.int32),
            pltpu.VMEM((BPW,), jnp.int32),
            pltpu.VMEM((BPW,), jnp.int32),
            pltpu.VMEM((7, SD), jnp.float32),
            pltpu.VMEM((24, SD), jnp.float32),
            pltpu.VMEM((12, SD), jnp.float32),
            pltpu.VMEM((31, SD), jnp.float32),
            pltpu.VMEM((D, 1), jnp.float32),
            pltpu.VMEM((1,), jnp.float32),
            pltpu.VMEM((BPW, D), jnp.float32),
            pltpu.VMEM((SP * L,), jnp.float32),
            pltpu.VMEM((BPW,), jnp.float32),
            pltpu.SemaphoreType.DMA,
            pltpu.SemaphoreType.DMA,
            pltpu.SemaphoreType.DMA,
            pltpu.SemaphoreType.DMA,
            pltpu.SemaphoreType.DMA,
            pltpu.SemaphoreType.DMA,
            pltpu.SemaphoreType.DMA,
        ],
        compiler_params=pltpu.CompilerParams(needs_layout_passes=False),
    )
    return f(dow, time, month, day, dest, W_dow, W_time, W_month, W_day,
             W_item, W_pred, b_pred)


def kernel(dayofweek, time, month, day, destination, W_dow, W_time,
           W_month, W_day, W_item, W_pred, b_pred):
    return _run(
        dayofweek.astype(jnp.int32), time.astype(jnp.int32),
        month.astype(jnp.int32), day.astype(jnp.int32),
        destination.astype(jnp.int32),
        W_dow, W_time, W_month, W_day, W_item, W_pred, b_pred)


# speedup vs baseline: 1.0581x; 1.0362x over previous
import jax
import jax.numpy as jnp
from jax import lax
from jax.experimental import pallas as pl
from jax.experimental.pallas import tpu as pltpu
from jax.experimental.pallas import tpu_sc as plsc

B = 16384
D = 128
SD = 32
NC = 2
NS = 16
NW = NC * NS
BPW = B // NW
GCH = 128
NG = BPW // GCH
L = 16
SP = L + 1


def _body(dow_h, time_h, month_h, day_h, dest_h,
          wdow_h, wtime_h, wmonth_h, wday_h, witem_h, wp_h, b_h,
          out_h,
          dest_v, dow_v, time_v, month_v, day_v,
          wdow_v, wtime_v, wmonth_v, wday_v, wp_v, b_v,
          item_v, stage_v, out_v,
          sem_s, sem_d, sem_o, sem_g0, sem_g1, sem_g2, sem_g3):
    wid = lax.axis_index("s") * NC + lax.axis_index("c")
    base = wid * BPW

    gsems = (sem_g0, sem_g1, sem_g2, sem_g3)
    pltpu.sync_copy(dest_h.at[pl.ds(base, GCH)], dest_v.at[0])
    gathers = [pltpu.async_copy(witem_h.at[dest_v.at[0]],
                                item_v.at[pl.ds(0, GCH)], gsems[0])]

    small = [
        pltpu.async_copy(dow_h.at[pl.ds(base, BPW)], dow_v, sem_s),
        pltpu.async_copy(time_h.at[pl.ds(base, BPW)], time_v, sem_s),
        pltpu.async_copy(month_h.at[pl.ds(base, BPW)], month_v, sem_s),
        pltpu.async_copy(day_h.at[pl.ds(base, BPW)], day_v, sem_s),
        pltpu.async_copy(wdow_h, wdow_v, sem_s),
        pltpu.async_copy(wtime_h, wtime_v, sem_s),
        pltpu.async_copy(wmonth_h, wmonth_v, sem_s),
        pltpu.async_copy(wday_h, wday_v, sem_s),
        pltpu.async_copy(wp_h, wp_v, sem_s),
        pltpu.async_copy(b_h, b_v, sem_s),
    ]

    for j in range(1, NG):
        pltpu.sync_copy(dest_h.at[pl.ds(base + j * GCH, GCH)], dest_v.at[j])
        gathers.append(
            pltpu.async_copy(witem_h.at[dest_v.at[j]],
                             item_v.at[pl.ds(j * GCH, GCH)], gsems[j]))

    for c in small:
        c.wait()

    lanes = lax.iota(jnp.int32, L)
    zeros = jnp.zeros((L,), jnp.int32)
    wp = [plsc.load_gather(wp_v, [lanes + L * j, zeros]) for j in range(D // L)]
    bvec = plsc.load_gather(b_v, [zeros])
    tables = (wdow_v, wtime_v, wmonth_v, wday_v)
    idx_refs = (dow_v, time_v, month_v, day_v)
    lanes_sp = lanes * SP

    for seg, (tab, nrows) in enumerate(zip(tables, (7, 24, 12, 31))):
        wpa, wpb = wp[2 * seg], wp[2 * seg + 1]

        def scale_body(r, carry, tab=tab, wpa=wpa, wpb=wpb):
            tab[r, pl.ds(0, L)] = tab[r, pl.ds(0, L)] * wpa
            tab[r, pl.ds(L, L)] = tab[r, pl.ds(L, L)] * wpb
            return carry

        lax.fori_loop(0, nrows, scale_body, 0)

    GPC = GCH // L

    def group_body(g, carry):
        @pl.when(g % GPC == 0)
        def _():
            for j in range(NG):
                @pl.when(g == j * GPC)
                def _(j=j):
                    gathers[j].wait()
        gbase = g * L
        ivecs = [r[pl.ds(gbase, L)] for r in idx_refs]
        RI = 4
        for rp in range(0, L, RI):
            accs = [None] * RI
            for j in range(D // L):
                seg, half = j // 2, (j % 2) * L
                for t in range(RI):
                    r = rp + t
                    it = item_v[gbase + r, pl.ds(L * j, L)]
                    us = tables[seg][ivecs[seg][r], pl.ds(half, L)]
                    term = it * us
                    accs[t] = term if accs[t] is None else accs[t] + term
            for t in range(RI):
                stage_v[pl.ds((rp + t) * SP, L)] = accs[t]
        cols = [plsc.load_gather(stage_v, [lanes_sp + c]) for c in range(L)]
        while len(cols) > 1:
            cols = [a + b for a, b in zip(cols[::2], cols[1::2])]
        x = cols[0] + bvec
        out_v[pl.ds(gbase, L)] = jnp.where(x >= 0, x, x * 0.01)
        return carry

    lax.fori_loop(0, BPW // L, group_body, 0)

    pltpu.sync_copy(out_v, out_h.at[pl.ds(base, BPW)])


@jax.jit
def _run(dow, time, month, day, dest, W_dow, W_time, W_month, W_day,
         W_item, W_pred, b_pred):
    mesh = plsc.VectorSubcoreMesh(core_axis_name="c", subcore_axis_name="s")
    f = pl.kernel(
        _body,
        out_type=jax.ShapeDtypeStruct((B,), jnp.float32),
        mesh=mesh,
        scratch_types=[
            pltpu.VMEM((NG, GCH), jnp.int32),
            pltpu.VMEM((BPW,), jnp.int32),
            pltpu.VMEM((BPW,), jnp.int32),
            pltpu.VMEM((BPW,), jnp.int32),
            pltpu.VMEM((BPW,), jnp.int32),
            pltpu.VMEM((7, SD), jnp.float32),
            pltpu.VMEM((24, SD), jnp.float32),
            pltpu.VMEM((12, SD), jnp.float32),
            pltpu.VMEM((31, SD), jnp.float32),
            pltpu.VMEM((D, 1), jnp.float32),
            pltpu.VMEM((1,), jnp.float32),
            pltpu.VMEM((BPW, D), jnp.float32),
            pltpu.VMEM((SP * L,), jnp.float32),
            pltpu.VMEM((BPW,), jnp.float32),
            pltpu.SemaphoreType.DMA,
            pltpu.SemaphoreType.DMA,
            pltpu.SemaphoreType.DMA,
            pltpu.SemaphoreType.DMA,
            pltpu.SemaphoreType.DMA,
            pltpu.SemaphoreType.DMA,
            pltpu.SemaphoreType.DMA,
        ],
        compiler_params=pltpu.CompilerParams(needs_layout_passes=False),
    )
    return f(dow, time, month, day, dest, W_dow, W_time, W_month, W_day,
             W_item, W_pred, b_pred)


def kernel(dayofweek, time, month, day, destination, W_dow, W_time,
           W_month, W_day, W_item, W_pred, b_pred):
    return _run(
        dayofweek.astype(jnp.int32), time.astype(jnp.int32),
        month.astype(jnp.int32), day.astype(jnp.int32),
        destination.astype(jnp.int32),
        W_dow, W_time, W_month, W_day, W_item, W_pred, b_pred)
